# Initial kernel scaffold; baseline (speedup 1.0000x reference)
import functools

import jax
import jax.numpy as jnp
from jax import lax
from jax.experimental import pallas as pl
from jax.experimental.pallas import tpu as pltpu
from jax.experimental.pallas import tpu_sc as plsc

N = 10000
E = 320000
D = 128
DP = 16

NC = 2
NS = 16
NW = NC * NS
L = 16

EW = E // NW
C = 80
NCH = EW // C

ZR = 125
RPT = N // NS

RPW = 313
C3 = 2560
NCH3 = E // C3

_LOG2 = 0.6931471805599453


def _ssp(x):
    return jnp.logaddexp(x, 0.0) - _LOG2


def _mesh():
    return plsc.VectorSubcoreMesh(
        core_axis_name="c", subcore_axis_name="s",
        num_cores=NC, num_subcores=NS)


def _zero_rows(ref, nrows):
    def body(i, _):
        for j in range(D // L):
            ref[i, pl.ds(j * L, L)] = jnp.zeros((L,), jnp.float32)
        return 0
    lax.fori_loop(0, nrows, body, 0)


def _zero_flat(ref, n):
    zv = jnp.zeros((L,), ref.dtype)
    def body(i, _):
        ref[pl.ds(i * L, L)] = zv
        return 0
    lax.fori_loop(0, n // L, body, 0)


def _sc_degrees(src, dst):
    NA = 10240
    NZ = NA // NS

    @functools.partial(
        pl.kernel,
        out_type=jax.ShapeDtypeStruct((NC, 2, N), jnp.float32),
        mesh=_mesh(),
        scratch_types=[
            pltpu.VMEM((C,), jnp.int32),
            pltpu.VMEM((C,), jnp.float32),
            pltpu.VMEM((NZ,), jnp.float32),
            pltpu.VMEM_SHARED((NA,), jnp.float32),
            pltpu.VMEM_SHARED((NA,), jnp.float32),
            pltpu.SemaphoreType.DMA,
        ],
    )
    def k(src_hbm, dst_hbm, out_hbm, idx_v, ones_v, zbuf, acc_s, acc_d, sem):
        c = lax.axis_index("c")
        s = lax.axis_index("s")
        w = c * NS + s

        _zero_flat(zbuf, NZ)
        ov = jnp.ones((L,), jnp.float32)
        def fill1(i, _):
            ones_v[pl.ds(i * L, L)] = ov
            return 0
        lax.fori_loop(0, C // L, fill1, 0)

        pltpu.sync_copy(zbuf, acc_s.at[pl.ds(s * NZ, NZ)])
        pltpu.sync_copy(zbuf, acc_d.at[pl.ds(s * NZ, NZ)])
        plsc.subcore_barrier()

        base0 = w * EW
        def body(kk, _):
            b = base0 + kk * C
            pltpu.sync_copy(src_hbm.at[pl.ds(b, C)], idx_v)
            pltpu.sync_copy(ones_v, acc_s.at[idx_v], add=True)
            pltpu.sync_copy(dst_hbm.at[pl.ds(b, C)], idx_v)
            pltpu.sync_copy(ones_v, acc_d.at[idx_v], add=True)
            return 0
        lax.fori_loop(0, NCH, body, 0)
        plsc.subcore_barrier()

        @pl.when(s < NS - 1)
        def _():
            pltpu.sync_copy(acc_s.at[pl.ds(s * NZ, NZ)], zbuf)
            pltpu.sync_copy(zbuf, out_hbm.at[c, 0, pl.ds(s * NZ, NZ)])
            pltpu.sync_copy(acc_d.at[pl.ds(s * NZ, NZ)], zbuf)
            pltpu.sync_copy(zbuf, out_hbm.at[c, 1, pl.ds(s * NZ, NZ)])

        @pl.when(s == NS - 1)
        def _():
            rem = N - (NS - 1) * NZ
            pltpu.sync_copy(acc_s.at[pl.ds((NS - 1) * NZ, rem)],
                            zbuf.at[pl.ds(0, rem)])
            pltpu.sync_copy(zbuf.at[pl.ds(0, rem)],
                            out_hbm.at[c, 0, pl.ds((NS - 1) * NZ, rem)])
            pltpu.sync_copy(acc_d.at[pl.ds((NS - 1) * NZ, rem)],
                            zbuf.at[pl.ds(0, rem)])
            pltpu.sync_copy(zbuf.at[pl.ds(0, rem)],
                            out_hbm.at[c, 1, pl.ds((NS - 1) * NZ, rem)])

    return k(src, dst)


def _sc_segsum(src, dst, table, wtab=None):
    have_w = wtab is not None
    scratch = [
        pltpu.VMEM((C,), jnp.int32),
        pltpu.VMEM((C,), jnp.int32),
        pltpu.VMEM((C, D), jnp.float32),
    ]
    if have_w:
        scratch.append(pltpu.VMEM((C, D), jnp.float32))
    scratch += [
        pltpu.VMEM((ZR, D), jnp.float32),
        pltpu.VMEM_SHARED((N, D), jnp.float32),
        pltpu.SemaphoreType.DMA,
    ]

    def body_fn(*refs):
        if have_w:
            (src_hbm, dst_hbm, tab_hbm, w_hbm, out_hbm,
             sidx, didx, rows, wrows, zstage, acc, sem) = refs
        else:
            (src_hbm, dst_hbm, tab_hbm, out_hbm,
             sidx, didx, rows, zstage, acc, sem) = refs
        c = lax.axis_index("c")
        s = lax.axis_index("s")
        w = c * NS + s

        _zero_rows(zstage, ZR)
        for t in range(RPT // ZR):
            pltpu.sync_copy(zstage, acc.at[pl.ds(s * RPT + t * ZR, ZR)])
        plsc.subcore_barrier()

        base0 = w * EW
        def body(kk, _):
            b = base0 + kk * C
            pltpu.sync_copy(src_hbm.at[pl.ds(b, C)], sidx)
            pltpu.sync_copy(dst_hbm.at[pl.ds(b, C)], didx)
            pltpu.async_copy(tab_hbm.at[sidx], rows, sem).wait()
            if have_w:
                pltpu.sync_copy(w_hbm.at[pl.ds(b, C)], wrows)
                def mull(i, _):
                    for j in range(D // L):
                        sl = pl.ds(j * L, L)
                        rows[i, sl] = rows[i, sl] * wrows[i, sl]
                    return 0
                lax.fori_loop(0, C, mull, 0)
            pltpu.sync_copy(rows, acc.at[didx], add=True)
            return 0
        lax.fori_loop(0, NCH, body, 0)
        plsc.subcore_barrier()

        for t in range(RPT // ZR):
            r0 = s * RPT + t * ZR
            pltpu.sync_copy(acc.at[pl.ds(r0, ZR)], zstage)
            pltpu.sync_copy(zstage, out_hbm.at[c, pl.ds(r0, ZR)])

    k = pl.kernel(
        body_fn,
        out_type=jax.ShapeDtypeStruct((NC, N, D), jnp.float32),
        mesh=_mesh(),
        scratch_types=scratch,
    )
    if have_w:
        return k(src, dst, table, wtab)
    return k(src, dst, table)


def _sc_segmax(src, dst, ew, table):
    @functools.partial(
        pl.kernel,
        out_type=jax.ShapeDtypeStruct((N * D,), jnp.float32),
        mesh=_mesh(),
        scratch_types=[
            pltpu.VMEM((C3,), jnp.int32),
            pltpu.VMEM((C3,), jnp.int32),
            pltpu.VMEM((C3,), jnp.float32),
            pltpu.VMEM((C3,), jnp.int32),
            pltpu.VMEM((C3,), jnp.int32),
            pltpu.VMEM((C3,), jnp.float32),
            pltpu.VMEM((L, D), jnp.float32),
            pltpu.VMEM((RPW * D,), jnp.float32),
            pltpu.SemaphoreType.DMA,
        ],
    )
    def k(src_hbm, dst_hbm, ew_hbm, tab_hbm, out_hbm,
          dbuf, sbuf, ebuf, msrc, mdst, mew, rows, acc, sem):
        c = lax.axis_index("c")
        s = lax.axis_index("s")
        w = c * NS + s
        lo = w * RPW
        hi = jnp.minimum(lo + RPW, N)

        _zero_flat(acc, RPW * D)
        _zero_flat(msrc, C3)
        _zero_flat(mdst, C3)

        def chunk(kk, _):
            b = kk * C3
            pltpu.sync_copy(dst_hbm.at[pl.ds(b, C3)], dbuf)
            pltpu.sync_copy(src_hbm.at[pl.ds(b, C3)], sbuf)
            pltpu.sync_copy(ew_hbm.at[pl.ds(b, C3)], ebuf)

            def scan(g, cnt):
                sl = pl.ds(g * L, L)
                d = dbuf[sl]
                m = (d >= lo) & (d < hi)
                plsc.store_compressed(msrc.at[pl.ds(cnt, L)], sbuf[sl], mask=m)
                plsc.store_compressed(mdst.at[pl.ds(cnt, L)], d - lo, mask=m)
                plsc.store_compressed(mew.at[pl.ds(cnt, L)], ebuf[sl], mask=m)
                return cnt + jnp.sum(m.astype(jnp.int32))
            cnt = lax.fori_loop(0, C3 // L, scan, jnp.int32(0))

            def grp(p, _):
                idxv = msrc[pl.ds(p * L, L)]
                pltpu.async_copy(tab_hbm.at[idxv], rows, sem).wait()
                vcnt = jnp.minimum(cnt - p * L, L)
                def edge(e, _):
                    dl = mdst[p * L + e]
                    wgt = mew[p * L + e]
                    for j in range(D // L):
                        sl2 = pl.ds(dl * D + j * L, L)
                        acc[sl2] = jnp.maximum(
                            acc[sl2], rows[e, pl.ds(j * L, L)] * wgt)
                    return 0
                lax.fori_loop(0, vcnt, edge, 0)
                return 0
            lax.fori_loop(0, (cnt + L - 1) // L, grp, 0)
            return 0
        lax.fori_loop(0, NCH3, chunk, 0)

        nlast = N - (NW - 1) * RPW
        @pl.when(w < NW - 1)
        def _():
            pltpu.sync_copy(acc, out_hbm.at[pl.ds(lo * D, RPW * D)])
        @pl.when(w == NW - 1)
        def _():
            pltpu.sync_copy(acc.at[pl.ds(0, nlast * D)],
                            out_hbm.at[pl.ds(lo * D, nlast * D)])

    return k(src, dst, ew, table)


def _tc_edge_dense(pin_feat, edge_feat, W_e1, b_e1, W_e2, b_e2, W_gw, b_gw):
    BE = 2000
    G = E // BE

    def body(pf, ef, w1, bb1, w2, bb2, wg, bg, he_o, ew_o):
        h = _ssp(jnp.dot(pf[...], w1[...],
                         preferred_element_type=jnp.float32) + bb1[...])
        he_o[...] = _ssp(jnp.dot(h, w2[...],
                                 preferred_element_type=jnp.float32) + bb2[...])
        ew_o[...] = jax.nn.sigmoid(
            jnp.dot(ef[...], wg[...],
                    preferred_element_type=jnp.float32) + bg[...])

    he, ew = pl.pallas_call(
        body,
        grid=(G,),
        in_specs=[
            pl.BlockSpec((BE, DP), lambda i: (i, 0)),
            pl.BlockSpec((BE, DP), lambda i: (i, 0)),
            pl.BlockSpec((DP, D), lambda i: (0, 0)),
            pl.BlockSpec((D,), lambda i: (0,)),
            pl.BlockSpec((D, D), lambda i: (0, 0)),
            pl.BlockSpec((D,), lambda i: (0,)),
            pl.BlockSpec((DP, 1), lambda i: (0, 0)),
            pl.BlockSpec((1,), lambda i: (0,)),
        ],
        out_specs=[
            pl.BlockSpec((BE, D), lambda i: (i, 0)),
            pl.BlockSpec((BE, 1), lambda i: (i, 0)),
        ],
        out_shape=[
            jax.ShapeDtypeStruct((E, D), jnp.float32),
            jax.ShapeDtypeStruct((E, 1), jnp.float32),
        ],
    )(pin_feat, edge_feat, W_e1, b_e1, W_e2, b_e2, W_gw, b_gw)
    return he, ew


def _tc_node_dense(net_feat, node_feat, W_n, b_n, W_pool, b_pool):
    BR = 2500
    G = N // BR

    def body(nf, xf, wn, bn, wp, bp, hv_o, fs_o):
        hv_o[...] = jnp.dot(nf[...], wn[...],
                            preferred_element_type=jnp.float32) + bn[...]
        fs_o[...] = jnp.maximum(
            jnp.dot(xf[...], wp[...],
                    preferred_element_type=jnp.float32) + bp[...], 0.0)

    hv, fs = pl.pallas_call(
        body,
        grid=(G,),
        in_specs=[
            pl.BlockSpec((BR, D), lambda i: (i, 0)),
            pl.BlockSpec((BR, D), lambda i: (i, 0)),
            pl.BlockSpec((D, D), lambda i: (0, 0)),
            pl.BlockSpec((D,), lambda i: (0,)),
            pl.BlockSpec((D, D), lambda i: (0, 0)),
            pl.BlockSpec((D,), lambda i: (0,)),
        ],
        out_specs=[
            pl.BlockSpec((BR, D), lambda i: (i, 0)),
            pl.BlockSpec((BR, D), lambda i: (i, 0)),
        ],
        out_shape=[
            jax.ShapeDtypeStruct((N, D), jnp.float32),
            jax.ShapeDtypeStruct((N, D), jnp.float32),
        ],
    )(net_feat, node_feat, W_n, b_n, W_pool, b_pool)
    return hv, fs


def _tc_scale(node_feat, degs):
    BR = 2500
    G = N // BR

    def body(xf, dg, o):
        d = jnp.maximum(dg[0] + dg[1], 1.0)
        o[...] = xf[...] * lax.rsqrt(d)[:, None]

    return pl.pallas_call(
        body,
        grid=(G,),
        in_specs=[
            pl.BlockSpec((BR, D), lambda i: (i, 0)),
            pl.BlockSpec((NC, BR), lambda i: (0, i)),
        ],
        out_specs=pl.BlockSpec((BR, D), lambda i: (i, 0)),
        out_shape=jax.ShapeDtypeStruct((N, D), jnp.float32),
    )(node_feat, degs)


def _tc_final(pins_parts, cf_parts, neigh, node_feat, deg_in_parts,
              W_pins, b_pins, W_o, b_o, W_self, b_self, W_neigh, b_neigh):
    BR = 2500
    G = N // BR

    def body(pp, cp, ng, nf, dp, wpi, bpi, wo, bo, ws, bs, wn, bn,
             hnode_o, hnet_o):
        agg = pp[0] + pp[1]
        di = jnp.maximum(dp[0] + dp[1], 1.0)
        aggn = agg * lax.rsqrt(di)[:, None]
        hnet_o[...] = jnp.dot(aggn, wpi[...],
                              preferred_element_type=jnp.float32) + bpi[...]
        agg2 = cp[0] + cp[1]
        out_cf = _ssp(jnp.dot(agg2, wo[...],
                              preferred_element_type=jnp.float32) + bo[...])
        out_sage = (jnp.dot(nf[...], ws[...],
                            preferred_element_type=jnp.float32) + bs[...]
                    + jnp.dot(ng[...], wn[...],
                              preferred_element_type=jnp.float32) + bn[...])
        hnode_o[...] = jnp.maximum(out_cf, out_sage)

    return pl.pallas_call(
        body,
        grid=(G,),
        in_specs=[
            pl.BlockSpec((NC, BR, D), lambda i: (0, i, 0)),
            pl.BlockSpec((NC, BR, D), lambda i: (0, i, 0)),
            pl.BlockSpec((BR, D), lambda i: (i, 0)),
            pl.BlockSpec((BR, D), lambda i: (i, 0)),
            pl.BlockSpec((NC, BR), lambda i: (0, i)),
            pl.BlockSpec((D, D), lambda i: (0, 0)),
            pl.BlockSpec((D,), lambda i: (0,)),
            pl.BlockSpec((D, D), lambda i: (0, 0)),
            pl.BlockSpec((D,), lambda i: (0,)),
            pl.BlockSpec((D, D), lambda i: (0, 0)),
            pl.BlockSpec((D,), lambda i: (0,)),
            pl.BlockSpec((D, D), lambda i: (0, 0)),
            pl.BlockSpec((D,), lambda i: (0,)),
        ],
        out_specs=[
            pl.BlockSpec((BR, D), lambda i: (i, 0)),
            pl.BlockSpec((BR, D), lambda i: (i, 0)),
        ],
        out_shape=[
            jax.ShapeDtypeStruct((N, D), jnp.float32),
            jax.ShapeDtypeStruct((N, D), jnp.float32),
        ],
    )(pins_parts, cf_parts, neigh, node_feat, deg_in_parts,
      W_pins, b_pins, W_o, b_o, W_self, b_self, W_neigh, b_neigh)


def kernel(node_feat, net_feat, pin_feat, edge_feat,
           pins_edge_index, pinned_edge_index, near_edge_index,
           W_gw, b_gw, W_pins, b_pins, W_e1, b_e1, W_e2, b_e2,
           W_n, b_n, W_o, b_o, W_pool, b_pool, W_self, b_self,
           W_neigh, b_neigh):
    src1, dst1 = pins_edge_index[0], pins_edge_index[1]
    src2, dst2 = pinned_edge_index[0], pinned_edge_index[1]
    src3, dst3 = near_edge_index[0], near_edge_index[1]

    he, ew2 = _tc_edge_dense(pin_feat, edge_feat, W_e1, b_e1, W_e2, b_e2,
                             W_gw, b_gw)
    ew = jnp.reshape(ew2, (E,))
    hv, feat_src = _tc_node_dense(net_feat, node_feat, W_n, b_n,
                                  W_pool, b_pool)

    degs = _sc_degrees(src1, dst1)
    feat = _tc_scale(node_feat, degs[:, 0])

    pins_parts = _sc_segsum(src1, dst1, feat)
    cf_parts = _sc_segsum(src2, dst2, hv, wtab=he)
    neigh = jnp.reshape(_sc_segmax(src3, dst3, ew, feat_src), (N, D))

    h_node, h_net = _tc_final(pins_parts, cf_parts, neigh, node_feat,
                              degs[:, 1], W_pins, b_pins, W_o, b_o,
                              W_self, b_self, W_neigh, b_neigh)
    return h_node, h_net


# speedup vs baseline: 1.4590x; 1.4590x over previous
import functools

import jax
import jax.numpy as jnp
from jax import lax
from jax.experimental import pallas as pl
from jax.experimental.pallas import tpu as pltpu
from jax.experimental.pallas import tpu_sc as plsc

N = 10000
E = 320000
D = 128
DP = 16

NC = 2
NS = 16
NW = NC * NS
L = 16

EW = E // NW
C = 80
NCH = EW // C

NP = 10240
ZR = 128
RPT = NP // NS

RPW = 313
C3 = 2560
NCH3 = E // C3

_LOG2 = 0.6931471805599453


def _ssp(x):
    return jnp.logaddexp(x, 0.0) - _LOG2


def _mesh():
    return plsc.VectorSubcoreMesh(
        core_axis_name="c", subcore_axis_name="s",
        num_cores=NC, num_subcores=NS)


def _zero_rows(ref, nrows):
    def body(i, _):
        for j in range(D // L):
            ref[i, pl.ds(j * L, L)] = jnp.zeros((L,), jnp.float32)
        return 0
    lax.fori_loop(0, nrows, body, 0)


def _zero_flat(ref, n):
    zv = jnp.zeros((L,), ref.dtype)
    def body(i, _):
        ref[pl.ds(i * L, L)] = zv
        return 0
    lax.fori_loop(0, n // L, body, 0)


def _sc_degrees(src, dst):
    NA = 10240
    NZ = NA // NS

    @functools.partial(
        pl.kernel,
        out_type=jax.ShapeDtypeStruct((NC, 2, NA), jnp.float32),
        mesh=_mesh(),
        scratch_types=[
            pltpu.VMEM((C,), jnp.int32),
            pltpu.VMEM((C,), jnp.float32),
            pltpu.VMEM((NZ,), jnp.float32),
            pltpu.VMEM_SHARED((NA,), jnp.float32),
            pltpu.VMEM_SHARED((NA,), jnp.float32),
            pltpu.SemaphoreType.DMA,
        ],
        compiler_params=pltpu.CompilerParams(needs_layout_passes=False),
    )
    def k(src_hbm, dst_hbm, out_hbm, idx_v, ones_v, zbuf, acc_s, acc_d, sem):
        c = lax.axis_index("c")
        s = lax.axis_index("s")
        w = c * NS + s

        _zero_flat(zbuf, NZ)
        ov = jnp.ones((L,), jnp.float32)
        def fill1(i, _):
            ones_v[pl.ds(i * L, L)] = ov
            return 0
        lax.fori_loop(0, C // L, fill1, 0)

        pltpu.sync_copy(zbuf, acc_s.at[pl.ds(s * NZ, NZ)])
        pltpu.sync_copy(zbuf, acc_d.at[pl.ds(s * NZ, NZ)])
        plsc.subcore_barrier()

        base0 = w * EW
        def body(kk, _):
            b = base0 + kk * C
            pltpu.sync_copy(src_hbm.at[pl.ds(b, C)], idx_v)
            pltpu.sync_copy(ones_v, acc_s.at[idx_v], add=True)
            pltpu.sync_copy(dst_hbm.at[pl.ds(b, C)], idx_v)
            pltpu.sync_copy(ones_v, acc_d.at[idx_v], add=True)
            return 0
        lax.fori_loop(0, NCH, body, 0)
        plsc.subcore_barrier()

        pltpu.sync_copy(acc_s.at[pl.ds(s * NZ, NZ)], zbuf)
        pltpu.sync_copy(zbuf, out_hbm.at[c, 0, pl.ds(s * NZ, NZ)])
        pltpu.sync_copy(acc_d.at[pl.ds(s * NZ, NZ)], zbuf)
        pltpu.sync_copy(zbuf, out_hbm.at[c, 1, pl.ds(s * NZ, NZ)])

    return k(src, dst)[:, :, :N]


def _sc_segsum(src, dst, table, wtab=None):
    have_w = wtab is not None
    scratch = [
        pltpu.VMEM((C,), jnp.int32),
        pltpu.VMEM((C,), jnp.int32),
        pltpu.VMEM((C, D), jnp.float32),
    ]
    if have_w:
        scratch.append(pltpu.VMEM((C, D), jnp.float32))
    scratch += [
        pltpu.VMEM((ZR, D), jnp.float32),
        pltpu.VMEM_SHARED((NP, D), jnp.float32),
        pltpu.SemaphoreType.DMA,
    ]

    def body_fn(*refs):
        if have_w:
            (src_hbm, dst_hbm, tab_hbm, w_hbm, out_hbm,
             sidx, didx, rows, wrows, zstage, acc, sem) = refs
        else:
            (src_hbm, dst_hbm, tab_hbm, out_hbm,
             sidx, didx, rows, zstage, acc, sem) = refs
        c = lax.axis_index("c")
        s = lax.axis_index("s")
        w = c * NS + s

        _zero_rows(zstage, ZR)
        for t in range(RPT // ZR):
            pltpu.sync_copy(zstage, acc.at[pl.ds(s * RPT + t * ZR, ZR)])
        plsc.subcore_barrier()

        base0 = w * EW
        def body(kk, _):
            b = base0 + kk * C
            pltpu.sync_copy(src_hbm.at[pl.ds(b, C)], sidx)
            pltpu.sync_copy(dst_hbm.at[pl.ds(b, C)], didx)
            pltpu.async_copy(tab_hbm.at[sidx], rows, sem).wait()
            if have_w:
                pltpu.sync_copy(w_hbm.at[pl.ds(b, C)], wrows)
                jv0 = lax.iota(jnp.int32, L)
                def mull(i, _):
                    iv = lax.broadcast(i, (L,))
                    for j in range(D // L):
                        jv = jv0 + (j * L)
                        a = plsc.load_gather(rows, [iv, jv])
                        bb = plsc.load_gather(wrows, [iv, jv])
                        plsc.store_scatter(rows, [iv, jv], a * bb)
                    return 0
                lax.fori_loop(0, C, mull, 0)
            pltpu.sync_copy(rows, acc.at[didx], add=True)
            return 0
        lax.fori_loop(0, NCH, body, 0)
        plsc.subcore_barrier()

        for t in range(RPT // ZR):
            r0 = s * RPT + t * ZR
            pltpu.sync_copy(acc.at[pl.ds(r0, ZR)], zstage)
            pltpu.sync_copy(zstage, out_hbm.at[c, pl.ds(r0, ZR)])

    k = pl.kernel(
        body_fn,
        out_type=jax.ShapeDtypeStruct((NC, NP, D), jnp.float32),
        mesh=_mesh(),
        scratch_types=scratch,
        compiler_params=pltpu.CompilerParams(needs_layout_passes=False),
    )
    if have_w:
        return k(src, dst, table, wtab)[:, :N]
    return k(src, dst, table)[:, :N]


def _sc_segmax(src, dst, ew, table):
    @functools.partial(
        pl.kernel,
        out_type=jax.ShapeDtypeStruct((N * D,), jnp.float32),
        mesh=_mesh(),
        scratch_types=[
            pltpu.VMEM((C3,), jnp.int32),
            pltpu.VMEM((C3,), jnp.int32),
            pltpu.VMEM((C3,), jnp.float32),
            pltpu.VMEM((C3 + L,), jnp.int32),
            pltpu.VMEM((C3 + L,), jnp.int32),
            pltpu.VMEM((C3 + L,), jnp.float32),
            pltpu.VMEM((L, D), jnp.float32),
            pltpu.VMEM((RPW * D,), jnp.float32),
            pltpu.SemaphoreType.DMA,
        ],
        compiler_params=pltpu.CompilerParams(needs_layout_passes=False),
    )
    def k(src_hbm, dst_hbm, ew_hbm, tab_hbm, out_hbm,
          dbuf, sbuf, ebuf, msrc, mdst, mew, rows, acc, sem):
        c = lax.axis_index("c")
        s = lax.axis_index("s")
        w = c * NS + s
        lo = w * RPW
        hi = jnp.minimum(lo + RPW, N)

        _zero_flat(acc, RPW * D)
        _zero_flat(msrc, C3 + L)
        _zero_flat(mdst, C3 + L)

        def chunk(kk, _):
            b = kk * C3
            pltpu.sync_copy(dst_hbm.at[pl.ds(b, C3)], dbuf)
            pltpu.sync_copy(src_hbm.at[pl.ds(b, C3)], sbuf)
            pltpu.sync_copy(ew_hbm.at[pl.ds(b, C3)], ebuf)

            def scan(g, cnt):
                sl = pl.ds(g * L, L)
                d = dbuf[sl]
                m = (d >= lo) & (d < hi)
                incl = plsc.cumsum(m.astype(jnp.int32))
                pos = cnt + incl - 1
                plsc.store_scatter(msrc, [pos], sbuf[sl], mask=m)
                plsc.store_scatter(mdst, [pos], d - lo, mask=m)
                plsc.store_scatter(mew, [pos], ebuf[sl], mask=m)
                return cnt + jnp.sum(m.astype(jnp.int32))
            cnt = lax.fori_loop(0, C3 // L, scan, jnp.int32(0))

            def grp(p, _):
                idxv = msrc[pl.ds(p * L, L)]
                pltpu.async_copy(tab_hbm.at[idxv], rows, sem).wait()
                vcnt = jnp.minimum(cnt - p * L, L)
                def edge(e, _):
                    ii = lax.broadcast(p * L + e, (L,))
                    dl = plsc.load_gather(mdst, [ii])[0]
                    wgt = plsc.load_gather(mew, [ii])[0]
                    for j in range(D // L):
                        sl2 = pl.ds(dl * D + j * L, L)
                        acc[sl2] = jnp.maximum(
                            acc[sl2], rows[e, pl.ds(j * L, L)] * wgt)
                    return 0
                lax.fori_loop(0, vcnt, edge, 0)
                return 0
            lax.fori_loop(0, (cnt + L - 1) // L, grp, 0)
            return 0
        lax.fori_loop(0, NCH3, chunk, 0)

        nlast = N - (NW - 1) * RPW
        @pl.when(w < NW - 1)
        def _():
            pltpu.sync_copy(acc, out_hbm.at[pl.ds(lo * D, RPW * D)])
        @pl.when(w == NW - 1)
        def _():
            pltpu.sync_copy(acc.at[pl.ds(0, nlast * D)],
                            out_hbm.at[pl.ds(lo * D, nlast * D)])

    return k(src, dst, ew, table)


def _tc_edge_dense(pin_feat, edge_feat, W_e1, b_e1, W_e2, b_e2, W_gw, b_gw):
    BE = 2000
    G = E // BE

    def body(pf, ef, w1, bb1, w2, bb2, wg, bg, he_o, ew_o):
        h = _ssp(jnp.dot(pf[...], w1[...],
                         preferred_element_type=jnp.float32) + bb1[...])
        he_o[...] = _ssp(jnp.dot(h, w2[...],
                                 preferred_element_type=jnp.float32) + bb2[...])
        ew_o[...] = jax.nn.sigmoid(
            jnp.dot(ef[...], wg[...],
                    preferred_element_type=jnp.float32) + bg[...])

    he, ew = pl.pallas_call(
        body,
        grid=(G,),
        in_specs=[
            pl.BlockSpec((BE, DP), lambda i: (i, 0)),
            pl.BlockSpec((BE, DP), lambda i: (i, 0)),
            pl.BlockSpec((DP, D), lambda i: (0, 0)),
            pl.BlockSpec((D,), lambda i: (0,)),
            pl.BlockSpec((D, D), lambda i: (0, 0)),
            pl.BlockSpec((D,), lambda i: (0,)),
            pl.BlockSpec((DP, 1), lambda i: (0, 0)),
            pl.BlockSpec((1,), lambda i: (0,)),
        ],
        out_specs=[
            pl.BlockSpec((BE, D), lambda i: (i, 0)),
            pl.BlockSpec((BE, 1), lambda i: (i, 0)),
        ],
        out_shape=[
            jax.ShapeDtypeStruct((E, D), jnp.float32),
            jax.ShapeDtypeStruct((E, 1), jnp.float32),
        ],
    )(pin_feat, edge_feat, W_e1, b_e1, W_e2, b_e2, W_gw, b_gw)
    return he, ew


def _tc_node_dense(net_feat, node_feat, W_n, b_n, W_pool, b_pool):
    BR = 2000
    G = N // BR

    def body(nf, xf, wn, bn, wp, bp, hv_o, fs_o):
        hv_o[...] = jnp.dot(nf[...], wn[...],
                            preferred_element_type=jnp.float32) + bn[...]
        fs_o[...] = jnp.maximum(
            jnp.dot(xf[...], wp[...],
                    preferred_element_type=jnp.float32) + bp[...], 0.0)

    hv, fs = pl.pallas_call(
        body,
        grid=(G,),
        in_specs=[
            pl.BlockSpec((BR, D), lambda i: (i, 0)),
            pl.BlockSpec((BR, D), lambda i: (i, 0)),
            pl.BlockSpec((D, D), lambda i: (0, 0)),
            pl.BlockSpec((D,), lambda i: (0,)),
            pl.BlockSpec((D, D), lambda i: (0, 0)),
            pl.BlockSpec((D,), lambda i: (0,)),
        ],
        out_specs=[
            pl.BlockSpec((BR, D), lambda i: (i, 0)),
            pl.BlockSpec((BR, D), lambda i: (i, 0)),
        ],
        out_shape=[
            jax.ShapeDtypeStruct((N, D), jnp.float32),
            jax.ShapeDtypeStruct((N, D), jnp.float32),
        ],
    )(net_feat, node_feat, W_n, b_n, W_pool, b_pool)
    return hv, fs


def _tc_scale(node_feat, degs):
    BR = 2000
    G = N // BR

    def body(xf, dg, o):
        d = jnp.maximum(dg[0, :, 0] + dg[1, :, 0], 1.0)
        o[...] = xf[...] * lax.rsqrt(d)[:, None]

    return pl.pallas_call(
        body,
        grid=(G,),
        in_specs=[
            pl.BlockSpec((BR, D), lambda i: (i, 0)),
            pl.BlockSpec((NC, BR, 1), lambda i: (0, i, 0)),
        ],
        out_specs=pl.BlockSpec((BR, D), lambda i: (i, 0)),
        out_shape=jax.ShapeDtypeStruct((N, D), jnp.float32),
    )(node_feat, degs)


def _tc_final(pins_parts, cf_parts, neigh, node_feat, deg_in_parts,
              W_pins, b_pins, W_o, b_o, W_self, b_self, W_neigh, b_neigh):
    BR = 2000
    G = N // BR

    def body(pp, cp, ng, nf, dp, wpi, bpi, wo, bo, ws, bs, wn, bn,
             hnode_o, hnet_o):
        agg = pp[0] + pp[1]
        di = jnp.maximum(dp[0, :, 0] + dp[1, :, 0], 1.0)
        aggn = agg * lax.rsqrt(di)[:, None]
        hnet_o[...] = jnp.dot(aggn, wpi[...],
                              preferred_element_type=jnp.float32) + bpi[...]
        agg2 = cp[0] + cp[1]
        out_cf = _ssp(jnp.dot(agg2, wo[...],
                              preferred_element_type=jnp.float32) + bo[...])
        out_sage = (jnp.dot(nf[...], ws[...],
                            preferred_element_type=jnp.float32) + bs[...]
                    + jnp.dot(ng[...], wn[...],
                              preferred_element_type=jnp.float32) + bn[...])
        hnode_o[...] = jnp.maximum(out_cf, out_sage)

    return pl.pallas_call(
        body,
        grid=(G,),
        in_specs=[
            pl.BlockSpec((NC, BR, D), lambda i: (0, i, 0)),
            pl.BlockSpec((NC, BR, D), lambda i: (0, i, 0)),
            pl.BlockSpec((BR, D), lambda i: (i, 0)),
            pl.BlockSpec((BR, D), lambda i: (i, 0)),
            pl.BlockSpec((NC, BR, 1), lambda i: (0, i, 0)),
            pl.BlockSpec((D, D), lambda i: (0, 0)),
            pl.BlockSpec((D,), lambda i: (0,)),
            pl.BlockSpec((D, D), lambda i: (0, 0)),
            pl.BlockSpec((D,), lambda i: (0,)),
            pl.BlockSpec((D, D), lambda i: (0, 0)),
            pl.BlockSpec((D,), lambda i: (0,)),
            pl.BlockSpec((D, D), lambda i: (0, 0)),
            pl.BlockSpec((D,), lambda i: (0,)),
        ],
        out_specs=[
            pl.BlockSpec((BR, D), lambda i: (i, 0)),
            pl.BlockSpec((BR, D), lambda i: (i, 0)),
        ],
        out_shape=[
            jax.ShapeDtypeStruct((N, D), jnp.float32),
            jax.ShapeDtypeStruct((N, D), jnp.float32),
        ],
    )(pins_parts, cf_parts, neigh, node_feat, deg_in_parts,
      W_pins, b_pins, W_o, b_o, W_self, b_self, W_neigh, b_neigh)


def kernel(node_feat, net_feat, pin_feat, edge_feat,
           pins_edge_index, pinned_edge_index, near_edge_index,
           W_gw, b_gw, W_pins, b_pins, W_e1, b_e1, W_e2, b_e2,
           W_n, b_n, W_o, b_o, W_pool, b_pool, W_self, b_self,
           W_neigh, b_neigh):
    src1, dst1 = pins_edge_index[0], pins_edge_index[1]
    src2, dst2 = pinned_edge_index[0], pinned_edge_index[1]
    src3, dst3 = near_edge_index[0], near_edge_index[1]

    he, ew2 = _tc_edge_dense(pin_feat, edge_feat, W_e1, b_e1, W_e2, b_e2,
                             W_gw, b_gw)
    ew = jnp.reshape(ew2, (E,))
    hv, feat_src = _tc_node_dense(net_feat, node_feat, W_n, b_n,
                                  W_pool, b_pool)

    degs = _sc_degrees(src1, dst1)
    feat = _tc_scale(node_feat, degs[:, 0][..., None])

    pins_parts = _sc_segsum(src1, dst1, feat)
    cf_parts = _sc_segsum(src2, dst2, hv, wtab=he)
    neigh = jnp.reshape(_sc_segmax(src3, dst3, ew, feat_src), (N, D))

    h_node, h_net = _tc_final(pins_parts, cf_parts, neigh, node_feat,
                              degs[:, 1][..., None], W_pins, b_pins, W_o, b_o,
                              W_self, b_self, W_neigh, b_neigh)
    return h_node, h_net


# speedup vs baseline: 2.4127x; 1.6537x over previous
import functools

import jax
import jax.numpy as jnp
from jax import lax
from jax.experimental import pallas as pl
from jax.experimental.pallas import tpu as pltpu
from jax.experimental.pallas import tpu_sc as plsc

N = 10000
E = 320000
D = 128
DP = 16

NC = 2
NS = 16
NW = NC * NS
L = 16

EW = E // NW
C = 80
NCH = EW // C

NP = 10240
ZR = 32
RPT = NP // NS

RPW = 313
C3 = 2560
NCH3 = E // C3

_LOG2 = 0.6931471805599453


def _ssp(x):
    return jnp.logaddexp(x, 0.0) - _LOG2


def _mesh():
    return plsc.VectorSubcoreMesh(
        core_axis_name="c", subcore_axis_name="s",
        num_cores=NC, num_subcores=NS)


def _zero_rows(ref, nrows):
    def body(i, _):
        for j in range(D // L):
            ref[i, pl.ds(j * L, L)] = jnp.zeros((L,), jnp.float32)
        return 0
    lax.fori_loop(0, nrows, body, 0)


def _zero_flat(ref, n):
    zv = jnp.zeros((L,), ref.dtype)
    def body(i, _):
        ref[pl.ds(i * L, L)] = zv
        return 0
    lax.fori_loop(0, n // L, body, 0)


def _sc_degrees(src, dst):
    NA = 10240
    NZ = NA // NS

    @functools.partial(
        pl.kernel,
        out_type=jax.ShapeDtypeStruct((NC, 2, NA), jnp.float32),
        mesh=_mesh(),
        scratch_types=[
            pltpu.VMEM((C,), jnp.int32),
            pltpu.VMEM((C,), jnp.float32),
            pltpu.VMEM((NZ,), jnp.float32),
            pltpu.VMEM_SHARED((NA,), jnp.float32),
            pltpu.VMEM_SHARED((NA,), jnp.float32),
            pltpu.SemaphoreType.DMA,
        ],
        compiler_params=pltpu.CompilerParams(needs_layout_passes=False),
    )
    def k(src_hbm, dst_hbm, out_hbm, idx_v, ones_v, zbuf, acc_s, acc_d, sem):
        c = lax.axis_index("c")
        s = lax.axis_index("s")
        w = c * NS + s

        _zero_flat(zbuf, NZ)
        ov = jnp.ones((L,), jnp.float32)
        def fill1(i, _):
            ones_v[pl.ds(i * L, L)] = ov
            return 0
        lax.fori_loop(0, C // L, fill1, 0)

        pltpu.sync_copy(zbuf, acc_s.at[pl.ds(s * NZ, NZ)])
        pltpu.sync_copy(zbuf, acc_d.at[pl.ds(s * NZ, NZ)])
        plsc.subcore_barrier()

        base0 = w * EW
        def body(kk, _):
            b = base0 + kk * C
            pltpu.sync_copy(src_hbm.at[pl.ds(b, C)], idx_v)
            pltpu.sync_copy(ones_v, acc_s.at[idx_v], add=True)
            pltpu.sync_copy(dst_hbm.at[pl.ds(b, C)], idx_v)
            pltpu.sync_copy(ones_v, acc_d.at[idx_v], add=True)
            return 0
        lax.fori_loop(0, NCH, body, 0)
        plsc.subcore_barrier()

        pltpu.sync_copy(acc_s.at[pl.ds(s * NZ, NZ)], zbuf)
        pltpu.sync_copy(zbuf, out_hbm.at[c, 0, pl.ds(s * NZ, NZ)])
        pltpu.sync_copy(acc_d.at[pl.ds(s * NZ, NZ)], zbuf)
        pltpu.sync_copy(zbuf, out_hbm.at[c, 1, pl.ds(s * NZ, NZ)])

    return k(src, dst)[:, :, :N]


def _sc_segsum(src, dst, table, wtab=None):
    have_w = wtab is not None
    scratch = [
        pltpu.VMEM((C,), jnp.int32),
        pltpu.VMEM((C,), jnp.int32),
        pltpu.VMEM((C,), jnp.int32),
        pltpu.VMEM((C,), jnp.int32),
        pltpu.VMEM((C, D), jnp.float32),
        pltpu.VMEM((C, D), jnp.float32),
    ]
    if have_w:
        scratch += [pltpu.VMEM((C, D), jnp.float32),
                    pltpu.SemaphoreType.DMA]
    scratch += [
        pltpu.VMEM((ZR, D), jnp.float32),
        pltpu.VMEM_SHARED((NP, D), jnp.float32),
        pltpu.SemaphoreType.DMA,
        pltpu.SemaphoreType.DMA,
        pltpu.SemaphoreType.DMA,
        pltpu.SemaphoreType.DMA,
    ]

    def body_fn(*refs):
        if have_w:
            (src_hbm, dst_hbm, tab_hbm, w_hbm, out_hbm,
             sidx0, sidx1, didx0, didx1, rows0, rows1, wrows1b, semw,
             zstage, acc, semi0, semi1, semr0, semr1) = refs
            wrows = wrows1b
        else:
            (src_hbm, dst_hbm, tab_hbm, out_hbm,
             sidx0, sidx1, didx0, didx1, rows0, rows1,
             zstage, acc, semi0, semi1, semr0, semr1) = refs
            w_hbm = None
            wrows = None
        sidx = (sidx0, sidx1)
        didx = (didx0, didx1)
        rows = (rows0, rows1)
        semi = (semi0, semi1)
        semr = (semr0, semr1)
        c = lax.axis_index("c")
        s = lax.axis_index("s")
        w = c * NS + s
        base0 = w * EW

        def start_loads(kk, par):
            b = base0 + kk * C
            pltpu.async_copy(src_hbm.at[pl.ds(b, C)], sidx[par], semi[par])
            pltpu.async_copy(dst_hbm.at[pl.ds(b, C)], didx[par], semi[par])

        def wait_loads(kk, par):
            b = base0 + kk * C
            pltpu.make_async_copy(src_hbm.at[pl.ds(b, C)], sidx[par],
                                  semi[par]).wait()
            pltpu.make_async_copy(dst_hbm.at[pl.ds(b, C)], didx[par],
                                  semi[par]).wait()

        def start_wload(kk):
            b = base0 + kk * C
            pltpu.async_copy(w_hbm.at[pl.ds(b, C)], wrows, semw)

        def wait_wload(kk):
            b = base0 + kk * C
            pltpu.make_async_copy(w_hbm.at[pl.ds(b, C)], wrows, semw).wait()

        def start_gather(par):
            pltpu.async_copy(tab_hbm.at[sidx[par]], rows[par], semr[par])

        def wait_gather(par):
            pltpu.make_async_copy(tab_hbm.at[sidx[par]], rows[par],
                                  semr[par]).wait()

        def process(kk, par):
            if have_w:
                wait_wload(kk)
                def mull(i, _):
                    for j in range(D // L):
                        sl = pl.ds(j * L, L)
                        rows[par][i, sl] = rows[par][i, sl] * wrows[i, sl]
                    return 0
                lax.fori_loop(0, C, mull, 0)
            pltpu.sync_copy(rows[par], acc.at[didx[par]], add=True)
            if have_w:
                @pl.when(kk + 1 < NCH)
                def _():
                    start_wload(kk + 1)

        _zero_rows(zstage, ZR)
        for t in range(RPT // ZR):
            pltpu.sync_copy(zstage, acc.at[pl.ds(s * RPT + t * ZR, ZR)])
        plsc.subcore_barrier()

        start_loads(0, 0)
        if have_w:
            start_wload(0)
        wait_loads(0, 0)
        start_gather(0)
        start_loads(1, 1)

        def pair(t, _):
            for par in range(2):
                kk = 2 * t + par
                nxt = 1 - par
                wait_loads(kk + 1, nxt)
                wait_gather(par)
                start_gather(nxt)
                process(kk, par)
                @pl.when(kk + 2 < NCH)
                def _():
                    start_loads(kk + 2, par)
            return 0
        lax.fori_loop(0, (NCH - 1) // 2, pair, 0)
        wait_gather(0)
        process(NCH - 1, 0)
        plsc.subcore_barrier()

        for t in range(RPT // ZR):
            r0 = s * RPT + t * ZR
            pltpu.sync_copy(acc.at[pl.ds(r0, ZR)], zstage)
            pltpu.sync_copy(zstage, out_hbm.at[c, pl.ds(r0, ZR)])

    k = pl.kernel(
        body_fn,
        out_type=jax.ShapeDtypeStruct((NC, NP, D), jnp.float32),
        mesh=_mesh(),
        scratch_types=scratch,
        compiler_params=pltpu.CompilerParams(needs_layout_passes=False),
    )
    if have_w:
        return k(src, dst, table, wtab)[:, :N]
    return k(src, dst, table)[:, :N]


def _sc_segmax(src, dst, ew, table):
    @functools.partial(
        pl.kernel,
        out_type=jax.ShapeDtypeStruct((N * D,), jnp.float32),
        mesh=_mesh(),
        scratch_types=[
            pltpu.VMEM((C3,), jnp.int32),
            pltpu.VMEM((C3,), jnp.int32),
            pltpu.VMEM((C3,), jnp.int32),
            pltpu.VMEM((C3,), jnp.int32),
            pltpu.VMEM((C3,), jnp.float32),
            pltpu.VMEM((C3,), jnp.float32),
            pltpu.VMEM((C3 + L,), jnp.int32),
            pltpu.VMEM((C3 + L,), jnp.int32),
            pltpu.VMEM((C3 + L,), jnp.float32),
            pltpu.VMEM((L, D), jnp.float32),
            pltpu.VMEM((L, D), jnp.float32),
            pltpu.VMEM((RPW * D,), jnp.float32),
            pltpu.SemaphoreType.DMA,
            pltpu.SemaphoreType.DMA,
        ],
        compiler_params=pltpu.CompilerParams(needs_layout_passes=False),
    )
    def k(src_hbm, dst_hbm, ew_hbm, tab_hbm, out_hbm,
          dbuf0, dbuf1, sbuf0, sbuf1, ebuf0, ebuf1,
          msrc, mdst, mew, rowsa, rowsb, acc, semc, semg):
        c = lax.axis_index("c")
        s = lax.axis_index("s")
        w = c * NS + s
        lo = w * RPW
        hi = jnp.minimum(lo + RPW, N)
        dbuf = (dbuf0, dbuf1)
        sbuf = (sbuf0, sbuf1)
        ebuf = (ebuf0, ebuf1)
        rows2 = (rowsa, rowsb)

        _zero_flat(acc, RPW * D)
        _zero_flat(msrc, C3 + L)
        _zero_flat(mdst, C3 + L)

        def start_cloads(kk, par):
            b = kk * C3
            pltpu.async_copy(dst_hbm.at[pl.ds(b, C3)], dbuf[par], semc)
            pltpu.async_copy(src_hbm.at[pl.ds(b, C3)], sbuf[par], semc)
            pltpu.async_copy(ew_hbm.at[pl.ds(b, C3)], ebuf[par], semc)

        def wait_cloads(kk, par):
            b = kk * C3
            pltpu.make_async_copy(dst_hbm.at[pl.ds(b, C3)], dbuf[par],
                                  semc).wait()
            pltpu.make_async_copy(src_hbm.at[pl.ds(b, C3)], sbuf[par],
                                  semc).wait()
            pltpu.make_async_copy(ew_hbm.at[pl.ds(b, C3)], ebuf[par],
                                  semc).wait()

        def start_g(p, buf):
            idxv = msrc[pl.ds(p * L, L)]
            pltpu.async_copy(tab_hbm.at[idxv], buf, semg)

        def wait_g(p, buf):
            idxv = msrc[pl.ds(p * L, L)]
            pltpu.make_async_copy(tab_hbm.at[idxv], buf, semg).wait()

        def do_chunk(kk, par):
            wait_cloads(kk, par)
            db, sb, eb = dbuf[par], sbuf[par], ebuf[par]

            def scan4(g4, cnt):
                tot = cnt
                for u in range(4):
                    sl = pl.ds((g4 * 4 + u) * L, L)
                    d = db[sl]
                    m = (d >= lo) & (d < hi)
                    incl = plsc.cumsum(m.astype(jnp.int32))
                    pos = tot + incl - 1
                    plsc.store_scatter(msrc, [pos], sb[sl], mask=m)
                    plsc.store_scatter(mdst, [pos], d - lo, mask=m)
                    plsc.store_scatter(mew, [pos], eb[sl], mask=m)
                    tot = tot + incl[L - 1]
                return tot
            cnt = lax.fori_loop(0, C3 // L // 4, scan4, jnp.int32(0))

            @pl.when(kk + 1 < NCH3)
            def _():
                start_cloads(kk + 1, 1 - par)

            ngrp = (cnt + L - 1) // L

            @pl.when(ngrp > 0)
            def _():
                start_g(0, rows2[0])

            def gpair(t, _):
                for gpar in range(2):
                    p = 2 * t + gpar
                    @pl.when(p < ngrp)
                    def _():
                        @pl.when(p + 1 < ngrp)
                        def _():
                            start_g(p + 1, rows2[1 - gpar])
                        wait_g(p, rows2[gpar])
                        vcnt = jnp.minimum(cnt - p * L, L)
                        def edge(e, _):
                            ii = lax.broadcast(p * L + e, (L,))
                            dl = plsc.load_gather(mdst, [ii])[0]
                            wgt = plsc.load_gather(mew, [ii])[0]
                            for j in range(D // L):
                                sl2 = pl.ds(dl * D + j * L, L)
                                acc[sl2] = jnp.maximum(
                                    acc[sl2],
                                    rows2[gpar][e, pl.ds(j * L, L)] * wgt)
                            return 0
                        lax.fori_loop(0, vcnt, edge, 0)
                return 0
            lax.fori_loop(0, (ngrp + 1) // 2, gpair, 0)

        start_cloads(0, 0)
        def pair(t, _):
            for par in range(2):
                do_chunk(2 * t + par, par)
            return 0
        lax.fori_loop(0, (NCH3 - 1) // 2, pair, 0)
        do_chunk(NCH3 - 1, 0)

        nlast = N - (NW - 1) * RPW
        @pl.when(w < NW - 1)
        def _():
            pltpu.sync_copy(acc, out_hbm.at[pl.ds(lo * D, RPW * D)])
        @pl.when(w == NW - 1)
        def _():
            pltpu.sync_copy(acc.at[pl.ds(0, nlast * D)],
                            out_hbm.at[pl.ds(lo * D, nlast * D)])

    return k(src, dst, ew, table)


def _tc_edge_dense(pin_feat, edge_feat, W_e1, b_e1, W_e2, b_e2, W_gw, b_gw):
    BE = 2000
    G = E // BE

    def body(pf, ef, w1, bb1, w2, bb2, wg, bg, he_o, ew_o):
        h = _ssp(jnp.dot(pf[...], w1[...],
                         preferred_element_type=jnp.float32) + bb1[...])
        he_o[...] = _ssp(jnp.dot(h, w2[...],
                                 preferred_element_type=jnp.float32) + bb2[...])
        ew_o[...] = jax.nn.sigmoid(
            jnp.dot(ef[...], wg[...],
                    preferred_element_type=jnp.float32) + bg[...])

    he, ew = pl.pallas_call(
        body,
        grid=(G,),
        in_specs=[
            pl.BlockSpec((BE, DP), lambda i: (i, 0)),
            pl.BlockSpec((BE, DP), lambda i: (i, 0)),
            pl.BlockSpec((DP, D), lambda i: (0, 0)),
            pl.BlockSpec((D,), lambda i: (0,)),
            pl.BlockSpec((D, D), lambda i: (0, 0)),
            pl.BlockSpec((D,), lambda i: (0,)),
            pl.BlockSpec((DP, 1), lambda i: (0, 0)),
            pl.BlockSpec((1,), lambda i: (0,)),
        ],
        out_specs=[
            pl.BlockSpec((BE, D), lambda i: (i, 0)),
            pl.BlockSpec((BE, 1), lambda i: (i, 0)),
        ],
        out_shape=[
            jax.ShapeDtypeStruct((E, D), jnp.float32),
            jax.ShapeDtypeStruct((E, 1), jnp.float32),
        ],
    )(pin_feat, edge_feat, W_e1, b_e1, W_e2, b_e2, W_gw, b_gw)
    return he, ew


def _tc_node_dense(net_feat, node_feat, W_n, b_n, W_pool, b_pool):
    BR = 2000
    G = N // BR

    def body(nf, xf, wn, bn, wp, bp, hv_o, fs_o):
        hv_o[...] = jnp.dot(nf[...], wn[...],
                            preferred_element_type=jnp.float32) + bn[...]
        fs_o[...] = jnp.maximum(
            jnp.dot(xf[...], wp[...],
                    preferred_element_type=jnp.float32) + bp[...], 0.0)

    hv, fs = pl.pallas_call(
        body,
        grid=(G,),
        in_specs=[
            pl.BlockSpec((BR, D), lambda i: (i, 0)),
            pl.BlockSpec((BR, D), lambda i: (i, 0)),
            pl.BlockSpec((D, D), lambda i: (0, 0)),
            pl.BlockSpec((D,), lambda i: (0,)),
            pl.BlockSpec((D, D), lambda i: (0, 0)),
            pl.BlockSpec((D,), lambda i: (0,)),
        ],
        out_specs=[
            pl.BlockSpec((BR, D), lambda i: (i, 0)),
            pl.BlockSpec((BR, D), lambda i: (i, 0)),
        ],
        out_shape=[
            jax.ShapeDtypeStruct((N, D), jnp.float32),
            jax.ShapeDtypeStruct((N, D), jnp.float32),
        ],
    )(net_feat, node_feat, W_n, b_n, W_pool, b_pool)
    return hv, fs


def _tc_scale(node_feat, degs):
    BR = 2000
    G = N // BR

    def body(xf, dg, o):
        d = jnp.maximum(dg[0, :, 0] + dg[1, :, 0], 1.0)
        o[...] = xf[...] * lax.rsqrt(d)[:, None]

    return pl.pallas_call(
        body,
        grid=(G,),
        in_specs=[
            pl.BlockSpec((BR, D), lambda i: (i, 0)),
            pl.BlockSpec((NC, BR, 1), lambda i: (0, i, 0)),
        ],
        out_specs=pl.BlockSpec((BR, D), lambda i: (i, 0)),
        out_shape=jax.ShapeDtypeStruct((N, D), jnp.float32),
    )(node_feat, degs)


def _tc_final(pins_parts, cf_parts, neigh, node_feat, deg_in_parts,
              W_pins, b_pins, W_o, b_o, W_self, b_self, W_neigh, b_neigh):
    BR = 2000
    G = N // BR

    def body(pp, cp, ng, nf, dp, wpi, bpi, wo, bo, ws, bs, wn, bn,
             hnode_o, hnet_o):
        agg = pp[0] + pp[1]
        di = jnp.maximum(dp[0, :, 0] + dp[1, :, 0], 1.0)
        aggn = agg * lax.rsqrt(di)[:, None]
        hnet_o[...] = jnp.dot(aggn, wpi[...],
                              preferred_element_type=jnp.float32) + bpi[...]
        agg2 = cp[0] + cp[1]
        out_cf = _ssp(jnp.dot(agg2, wo[...],
                              preferred_element_type=jnp.float32) + bo[...])
        out_sage = (jnp.dot(nf[...], ws[...],
                            preferred_element_type=jnp.float32) + bs[...]
                    + jnp.dot(ng[...], wn[...],
                              preferred_element_type=jnp.float32) + bn[...])
        hnode_o[...] = jnp.maximum(out_cf, out_sage)

    return pl.pallas_call(
        body,
        grid=(G,),
        in_specs=[
            pl.BlockSpec((NC, BR, D), lambda i: (0, i, 0)),
            pl.BlockSpec((NC, BR, D), lambda i: (0, i, 0)),
            pl.BlockSpec((BR, D), lambda i: (i, 0)),
            pl.BlockSpec((BR, D), lambda i: (i, 0)),
            pl.BlockSpec((NC, BR, 1), lambda i: (0, i, 0)),
            pl.BlockSpec((D, D), lambda i: (0, 0)),
            pl.BlockSpec((D,), lambda i: (0,)),
            pl.BlockSpec((D, D), lambda i: (0, 0)),
            pl.BlockSpec((D,), lambda i: (0,)),
            pl.BlockSpec((D, D), lambda i: (0, 0)),
            pl.BlockSpec((D,), lambda i: (0,)),
            pl.BlockSpec((D, D), lambda i: (0, 0)),
            pl.BlockSpec((D,), lambda i: (0,)),
        ],
        out_specs=[
            pl.BlockSpec((BR, D), lambda i: (i, 0)),
            pl.BlockSpec((BR, D), lambda i: (i, 0)),
        ],
        out_shape=[
            jax.ShapeDtypeStruct((N, D), jnp.float32),
            jax.ShapeDtypeStruct((N, D), jnp.float32),
        ],
    )(pins_parts, cf_parts, neigh, node_feat, deg_in_parts,
      W_pins, b_pins, W_o, b_o, W_self, b_self, W_neigh, b_neigh)


def kernel(node_feat, net_feat, pin_feat, edge_feat,
           pins_edge_index, pinned_edge_index, near_edge_index,
           W_gw, b_gw, W_pins, b_pins, W_e1, b_e1, W_e2, b_e2,
           W_n, b_n, W_o, b_o, W_pool, b_pool, W_self, b_self,
           W_neigh, b_neigh):
    src1, dst1 = pins_edge_index[0], pins_edge_index[1]
    src2, dst2 = pinned_edge_index[0], pinned_edge_index[1]
    src3, dst3 = near_edge_index[0], near_edge_index[1]

    he, ew2 = _tc_edge_dense(pin_feat, edge_feat, W_e1, b_e1, W_e2, b_e2,
                             W_gw, b_gw)
    ew = jnp.reshape(ew2, (E,))
    hv, feat_src = _tc_node_dense(net_feat, node_feat, W_n, b_n,
                                  W_pool, b_pool)

    degs = _sc_degrees(src1, dst1)
    feat = _tc_scale(node_feat, degs[:, 0][..., None])

    pins_parts = _sc_segsum(src1, dst1, feat)
    cf_parts = _sc_segsum(src2, dst2, hv, wtab=he)
    neigh = jnp.reshape(_sc_segmax(src3, dst3, ew, feat_src), (N, D))

    h_node, h_net = _tc_final(pins_parts, cf_parts, neigh, node_feat,
                              degs[:, 1][..., None], W_pins, b_pins, W_o, b_o,
                              W_self, b_self, W_neigh, b_neigh)
    return h_node, h_net


# speedup vs baseline: 2.4128x; 1.0000x over previous
import functools

import jax
import jax.numpy as jnp
from jax import lax
from jax.experimental import pallas as pl
from jax.experimental.pallas import tpu as pltpu
from jax.experimental.pallas import tpu_sc as plsc

N = 10000
E = 320000
D = 128
DP = 16

NC = 2
NS = 16
NW = NC * NS
L = 16

EW = E // NW
C = 80
NCH = EW // C

NP = 10240
ZR = 32
RPT = NP // NS

RPW = 313
C3 = 2560
NCH3 = E // C3

_LOG2 = 0.6931471805599453


def _ssp(x):
    return jnp.logaddexp(x, 0.0) - _LOG2


def _mesh():
    return plsc.VectorSubcoreMesh(
        core_axis_name="c", subcore_axis_name="s",
        num_cores=NC, num_subcores=NS)


def _zero_rows(ref, nrows):
    def body(i, _):
        for j in range(D // L):
            ref[i, pl.ds(j * L, L)] = jnp.zeros((L,), jnp.float32)
        return 0
    lax.fori_loop(0, nrows, body, 0)


def _zero_flat(ref, n):
    zv = jnp.zeros((L,), ref.dtype)
    def body(i, _):
        ref[pl.ds(i * L, L)] = zv
        return 0
    lax.fori_loop(0, n // L, body, 0)


def _sc_degrees(src, dst):
    NA = 10240
    NZ = NA // NS

    @functools.partial(
        pl.kernel,
        out_type=jax.ShapeDtypeStruct((NC, 2, NA), jnp.float32),
        mesh=_mesh(),
        scratch_types=[
            pltpu.VMEM((C,), jnp.int32),
            pltpu.VMEM((C,), jnp.int32),
            pltpu.VMEM((C,), jnp.int32),
            pltpu.VMEM((C,), jnp.int32),
            pltpu.VMEM((C,), jnp.float32),
            pltpu.VMEM((NZ,), jnp.float32),
            pltpu.VMEM_SHARED((NA,), jnp.float32),
            pltpu.VMEM_SHARED((NA,), jnp.float32),
            pltpu.SemaphoreType.DMA,
            pltpu.SemaphoreType.DMA,
        ],
        compiler_params=pltpu.CompilerParams(needs_layout_passes=False),
    )
    def k(src_hbm, dst_hbm, out_hbm, sidx0, sidx1, didx0, didx1,
          ones_v, zbuf, acc_s, acc_d, semi0, semi1):
        sidxb = (sidx0, sidx1)
        didxb = (didx0, didx1)
        semi = (semi0, semi1)
        c = lax.axis_index("c")
        s = lax.axis_index("s")
        w = c * NS + s

        _zero_flat(zbuf, NZ)
        ov = jnp.ones((L,), jnp.float32)
        def fill1(i, _):
            ones_v[pl.ds(i * L, L)] = ov
            return 0
        lax.fori_loop(0, C // L, fill1, 0)

        pltpu.sync_copy(zbuf, acc_s.at[pl.ds(s * NZ, NZ)])
        pltpu.sync_copy(zbuf, acc_d.at[pl.ds(s * NZ, NZ)])
        plsc.subcore_barrier()

        base0 = w * EW

        def dstart(kk, par):
            b = base0 + kk * C
            pltpu.async_copy(src_hbm.at[pl.ds(b, C)], sidxb[par], semi[par])
            pltpu.async_copy(dst_hbm.at[pl.ds(b, C)], didxb[par], semi[par])

        def dwait(kk, par):
            b = base0 + kk * C
            pltpu.make_async_copy(src_hbm.at[pl.ds(b, C)], sidxb[par],
                                  semi[par]).wait()
            pltpu.make_async_copy(dst_hbm.at[pl.ds(b, C)], didxb[par],
                                  semi[par]).wait()

        dstart(0, 0)
        dstart(1, 1)
        def dpair(t, _):
            for par in range(2):
                kk = 2 * t + par
                dwait(kk, par)
                pltpu.sync_copy(ones_v, acc_s.at[sidxb[par]], add=True)
                pltpu.sync_copy(ones_v, acc_d.at[didxb[par]], add=True)
                @pl.when(kk + 2 < NCH)
                def _():
                    dstart(kk + 2, par)
            return 0
        lax.fori_loop(0, NCH // 2, dpair, 0)
        dwait(NCH - 1, 0)
        pltpu.sync_copy(ones_v, acc_s.at[sidxb[0]], add=True)
        pltpu.sync_copy(ones_v, acc_d.at[didxb[0]], add=True)
        plsc.subcore_barrier()

        pltpu.sync_copy(acc_s.at[pl.ds(s * NZ, NZ)], zbuf)
        pltpu.sync_copy(zbuf, out_hbm.at[c, 0, pl.ds(s * NZ, NZ)])
        pltpu.sync_copy(acc_d.at[pl.ds(s * NZ, NZ)], zbuf)
        pltpu.sync_copy(zbuf, out_hbm.at[c, 1, pl.ds(s * NZ, NZ)])

    return k(src, dst)[:, :, :N]


def _sc_segsum(src, dst, table, wtab=None):
    have_w = wtab is not None
    scratch = [
        pltpu.VMEM((C,), jnp.int32),
        pltpu.VMEM((C,), jnp.int32),
        pltpu.VMEM((C,), jnp.int32),
        pltpu.VMEM((C,), jnp.int32),
        pltpu.VMEM((C, D), jnp.float32),
        pltpu.VMEM((C, D), jnp.float32),
    ]
    if have_w:
        scratch += [pltpu.VMEM((C, D), jnp.float32),
                    pltpu.SemaphoreType.DMA]
    scratch += [
        pltpu.VMEM((ZR, D), jnp.float32),
        pltpu.VMEM_SHARED((NP, D), jnp.float32),
        pltpu.SemaphoreType.DMA,
        pltpu.SemaphoreType.DMA,
        pltpu.SemaphoreType.DMA,
        pltpu.SemaphoreType.DMA,
    ]

    def body_fn(*refs):
        if have_w:
            (src_hbm, dst_hbm, tab_hbm, w_hbm, out_hbm,
             sidx0, sidx1, didx0, didx1, rows0, rows1, wrows1b, semw,
             zstage, acc, semi0, semi1, semr0, semr1) = refs
            wrows = wrows1b
        else:
            (src_hbm, dst_hbm, tab_hbm, out_hbm,
             sidx0, sidx1, didx0, didx1, rows0, rows1,
             zstage, acc, semi0, semi1, semr0, semr1) = refs
            w_hbm = None
            wrows = None
        sidx = (sidx0, sidx1)
        didx = (didx0, didx1)
        rows = (rows0, rows1)
        semi = (semi0, semi1)
        semr = (semr0, semr1)
        c = lax.axis_index("c")
        s = lax.axis_index("s")
        w = c * NS + s
        base0 = w * EW

        def start_loads(kk, par):
            b = base0 + kk * C
            pltpu.async_copy(src_hbm.at[pl.ds(b, C)], sidx[par], semi[par])
            pltpu.async_copy(dst_hbm.at[pl.ds(b, C)], didx[par], semi[par])

        def wait_loads(kk, par):
            b = base0 + kk * C
            pltpu.make_async_copy(src_hbm.at[pl.ds(b, C)], sidx[par],
                                  semi[par]).wait()
            pltpu.make_async_copy(dst_hbm.at[pl.ds(b, C)], didx[par],
                                  semi[par]).wait()

        def start_wload(kk):
            b = base0 + kk * C
            pltpu.async_copy(w_hbm.at[pl.ds(b, C)], wrows, semw)

        def wait_wload(kk):
            b = base0 + kk * C
            pltpu.make_async_copy(w_hbm.at[pl.ds(b, C)], wrows, semw).wait()

        def start_gather(par):
            pltpu.async_copy(tab_hbm.at[sidx[par]], rows[par], semr[par])

        def wait_gather(par):
            pltpu.make_async_copy(tab_hbm.at[sidx[par]], rows[par],
                                  semr[par]).wait()

        def process(kk, par):
            if have_w:
                wait_wload(kk)
                def mull(i, _):
                    for j in range(D // L):
                        sl = pl.ds(j * L, L)
                        rows[par][i, sl] = rows[par][i, sl] * wrows[i, sl]
                    return 0
                lax.fori_loop(0, C, mull, 0)
            pltpu.sync_copy(rows[par], acc.at[didx[par]], add=True)
            if have_w:
                @pl.when(kk + 1 < NCH)
                def _():
                    start_wload(kk + 1)

        _zero_rows(zstage, ZR)
        for t in range(RPT // ZR):
            pltpu.sync_copy(zstage, acc.at[pl.ds(s * RPT + t * ZR, ZR)])
        plsc.subcore_barrier()

        start_loads(0, 0)
        if have_w:
            start_wload(0)
        wait_loads(0, 0)
        start_gather(0)
        start_loads(1, 1)

        def pair(t, _):
            for par in range(2):
                kk = 2 * t + par
                nxt = 1 - par
                wait_loads(kk + 1, nxt)
                wait_gather(par)
                start_gather(nxt)
                process(kk, par)
                @pl.when(kk + 2 < NCH)
                def _():
                    start_loads(kk + 2, par)
            return 0
        lax.fori_loop(0, (NCH - 1) // 2, pair, 0)
        wait_gather(0)
        process(NCH - 1, 0)
        plsc.subcore_barrier()

        for t in range(RPT // ZR):
            r0 = s * RPT + t * ZR
            pltpu.sync_copy(acc.at[pl.ds(r0, ZR)], zstage)
            pltpu.sync_copy(zstage, out_hbm.at[c, pl.ds(r0, ZR)])

    k = pl.kernel(
        body_fn,
        out_type=jax.ShapeDtypeStruct((NC, NP, D), jnp.float32),
        mesh=_mesh(),
        scratch_types=scratch,
        compiler_params=pltpu.CompilerParams(needs_layout_passes=False),
    )
    if have_w:
        return k(src, dst, table, wtab)[:, :N]
    return k(src, dst, table)[:, :N]


def _sc_segmax(src, dst, ew, table):
    @functools.partial(
        pl.kernel,
        out_type=jax.ShapeDtypeStruct((N * D,), jnp.float32),
        mesh=_mesh(),
        scratch_types=[
            pltpu.VMEM((C3,), jnp.int32),
            pltpu.VMEM((C3,), jnp.int32),
            pltpu.VMEM((C3,), jnp.int32),
            pltpu.VMEM((C3,), jnp.int32),
            pltpu.VMEM((C3,), jnp.float32),
            pltpu.VMEM((C3,), jnp.float32),
            pltpu.VMEM((C3 + L,), jnp.int32),
            pltpu.VMEM((C3 + L,), jnp.int32),
            pltpu.VMEM((C3 + L,), jnp.float32),
            pltpu.VMEM((L, D), jnp.float32),
            pltpu.VMEM((L, D), jnp.float32),
            pltpu.VMEM((RPW * D,), jnp.float32),
            pltpu.SemaphoreType.DMA,
            pltpu.SemaphoreType.DMA,
        ],
        compiler_params=pltpu.CompilerParams(needs_layout_passes=False),
    )
    def k(src_hbm, dst_hbm, ew_hbm, tab_hbm, out_hbm,
          dbuf0, dbuf1, sbuf0, sbuf1, ebuf0, ebuf1,
          msrc, mdst, mew, rowsa, rowsb, acc, semc, semg):
        c = lax.axis_index("c")
        s = lax.axis_index("s")
        w = c * NS + s
        lo = w * RPW
        hi = jnp.minimum(lo + RPW, N)
        dbuf = (dbuf0, dbuf1)
        sbuf = (sbuf0, sbuf1)
        ebuf = (ebuf0, ebuf1)
        rows2 = (rowsa, rowsb)

        _zero_flat(acc, RPW * D)
        _zero_flat(msrc, C3 + L)
        _zero_flat(mdst, C3 + L)

        def start_cloads(kk, par):
            b = kk * C3
            pltpu.async_copy(dst_hbm.at[pl.ds(b, C3)], dbuf[par], semc)
            pltpu.async_copy(src_hbm.at[pl.ds(b, C3)], sbuf[par], semc)
            pltpu.async_copy(ew_hbm.at[pl.ds(b, C3)], ebuf[par], semc)

        def wait_cloads(kk, par):
            b = kk * C3
            pltpu.make_async_copy(dst_hbm.at[pl.ds(b, C3)], dbuf[par],
                                  semc).wait()
            pltpu.make_async_copy(src_hbm.at[pl.ds(b, C3)], sbuf[par],
                                  semc).wait()
            pltpu.make_async_copy(ew_hbm.at[pl.ds(b, C3)], ebuf[par],
                                  semc).wait()

        def start_g(p, buf):
            idxv = msrc[pl.ds(p * L, L)]
            pltpu.async_copy(tab_hbm.at[idxv], buf, semg)

        def wait_g(p, buf):
            idxv = msrc[pl.ds(p * L, L)]
            pltpu.make_async_copy(tab_hbm.at[idxv], buf, semg).wait()

        def do_chunk(kk, par):
            wait_cloads(kk, par)
            db, sb, eb = dbuf[par], sbuf[par], ebuf[par]

            nseg = hi - lo
            def scan4(g4, cntv):
                tot = cntv
                for u in range(4):
                    sl = pl.ds((g4 * 4 + u) * L, L)
                    d = db[sl]
                    dl = d - lo
                    m = dl.astype(jnp.uint32) < nseg.astype(jnp.uint32)
                    incl = plsc.cumsum(m.astype(jnp.int32))
                    pos = tot + incl - 1
                    plsc.store_scatter(msrc, [pos], sb[sl], mask=m)
                    plsc.store_scatter(mdst, [pos], dl, mask=m)
                    plsc.store_scatter(mew, [pos], eb[sl], mask=m)
                    tot = tot + plsc.all_reduce_population_count(m)
                return tot
            cntv = lax.fori_loop(0, C3 // L // 4, scan4,
                                 jnp.zeros((L,), jnp.int32))
            cnt = cntv[0]

            @pl.when(kk + 1 < NCH3)
            def _():
                start_cloads(kk + 1, 1 - par)

            ngrp = (cnt + L - 1) // L

            @pl.when(ngrp > 0)
            def _():
                start_g(0, rows2[0])

            def gpair(t, _):
                for gpar in range(2):
                    p = 2 * t + gpar
                    @pl.when(p < ngrp)
                    def _():
                        @pl.when(p + 1 < ngrp)
                        def _():
                            start_g(p + 1, rows2[1 - gpar])
                        wait_g(p, rows2[gpar])
                        vcnt = jnp.minimum(cnt - p * L, L)
                        def edge(e, _):
                            ii = lax.broadcast(p * L + e, (L,))
                            dl = plsc.load_gather(mdst, [ii])[0]
                            wgt = plsc.load_gather(mew, [ii])[0]
                            for j in range(D // L):
                                sl2 = pl.ds(dl * D + j * L, L)
                                acc[sl2] = jnp.maximum(
                                    acc[sl2],
                                    rows2[gpar][e, pl.ds(j * L, L)] * wgt)
                            return 0
                        lax.fori_loop(0, vcnt, edge, 0)
                return 0
            lax.fori_loop(0, (ngrp + 1) // 2, gpair, 0)

        start_cloads(0, 0)
        def pair(t, _):
            for par in range(2):
                do_chunk(2 * t + par, par)
            return 0
        lax.fori_loop(0, (NCH3 - 1) // 2, pair, 0)
        do_chunk(NCH3 - 1, 0)

        nlast = N - (NW - 1) * RPW
        @pl.when(w < NW - 1)
        def _():
            pltpu.sync_copy(acc, out_hbm.at[pl.ds(lo * D, RPW * D)])
        @pl.when(w == NW - 1)
        def _():
            pltpu.sync_copy(acc.at[pl.ds(0, nlast * D)],
                            out_hbm.at[pl.ds(lo * D, nlast * D)])

    return k(src, dst, ew, table)


def _tc_edge_dense(pin_feat, edge_feat, W_e1, b_e1, W_e2, b_e2, W_gw, b_gw):
    BE = 2000
    G = E // BE

    def body(pf, ef, w1, bb1, w2, bb2, wg, bg, he_o, ew_o):
        h = _ssp(jnp.dot(pf[...], w1[...],
                         preferred_element_type=jnp.float32) + bb1[...])
        he_o[...] = _ssp(jnp.dot(h, w2[...],
                                 preferred_element_type=jnp.float32) + bb2[...])
        ew_o[...] = jax.nn.sigmoid(
            jnp.dot(ef[...], wg[...],
                    preferred_element_type=jnp.float32) + bg[...])

    he, ew = pl.pallas_call(
        body,
        grid=(G,),
        in_specs=[
            pl.BlockSpec((BE, DP), lambda i: (i, 0)),
            pl.BlockSpec((BE, DP), lambda i: (i, 0)),
            pl.BlockSpec((DP, D), lambda i: (0, 0)),
            pl.BlockSpec((D,), lambda i: (0,)),
            pl.BlockSpec((D, D), lambda i: (0, 0)),
            pl.BlockSpec((D,), lambda i: (0,)),
            pl.BlockSpec((DP, 1), lambda i: (0, 0)),
            pl.BlockSpec((1,), lambda i: (0,)),
        ],
        out_specs=[
            pl.BlockSpec((BE, D), lambda i: (i, 0)),
            pl.BlockSpec((BE, 1), lambda i: (i, 0)),
        ],
        out_shape=[
            jax.ShapeDtypeStruct((E, D), jnp.float32),
            jax.ShapeDtypeStruct((E, 1), jnp.float32),
        ],
    )(pin_feat, edge_feat, W_e1, b_e1, W_e2, b_e2, W_gw, b_gw)
    return he, ew


def _tc_node_dense(net_feat, node_feat, W_n, b_n, W_pool, b_pool):
    BR = 2000
    G = N // BR

    def body(nf, xf, wn, bn, wp, bp, hv_o, fs_o):
        hv_o[...] = jnp.dot(nf[...], wn[...],
                            preferred_element_type=jnp.float32) + bn[...]
        fs_o[...] = jnp.maximum(
            jnp.dot(xf[...], wp[...],
                    preferred_element_type=jnp.float32) + bp[...], 0.0)

    hv, fs = pl.pallas_call(
        body,
        grid=(G,),
        in_specs=[
            pl.BlockSpec((BR, D), lambda i: (i, 0)),
            pl.BlockSpec((BR, D), lambda i: (i, 0)),
            pl.BlockSpec((D, D), lambda i: (0, 0)),
            pl.BlockSpec((D,), lambda i: (0,)),
            pl.BlockSpec((D, D), lambda i: (0, 0)),
            pl.BlockSpec((D,), lambda i: (0,)),
        ],
        out_specs=[
            pl.BlockSpec((BR, D), lambda i: (i, 0)),
            pl.BlockSpec((BR, D), lambda i: (i, 0)),
        ],
        out_shape=[
            jax.ShapeDtypeStruct((N, D), jnp.float32),
            jax.ShapeDtypeStruct((N, D), jnp.float32),
        ],
    )(net_feat, node_feat, W_n, b_n, W_pool, b_pool)
    return hv, fs


def _tc_scale(node_feat, degs):
    BR = 2000
    G = N // BR

    def body(xf, dg, o):
        d = jnp.maximum(dg[0, :, 0] + dg[1, :, 0], 1.0)
        o[...] = xf[...] * lax.rsqrt(d)[:, None]

    return pl.pallas_call(
        body,
        grid=(G,),
        in_specs=[
            pl.BlockSpec((BR, D), lambda i: (i, 0)),
            pl.BlockSpec((NC, BR, 1), lambda i: (0, i, 0)),
        ],
        out_specs=pl.BlockSpec((BR, D), lambda i: (i, 0)),
        out_shape=jax.ShapeDtypeStruct((N, D), jnp.float32),
    )(node_feat, degs)


def _tc_final(pins_parts, cf_parts, neigh, node_feat, deg_in_parts,
              W_pins, b_pins, W_o, b_o, W_self, b_self, W_neigh, b_neigh):
    BR = 2000
    G = N // BR

    def body(pp, cp, ng, nf, dp, wpi, bpi, wo, bo, ws, bs, wn, bn,
             hnode_o, hnet_o):
        agg = pp[0] + pp[1]
        di = jnp.maximum(dp[0, :, 0] + dp[1, :, 0], 1.0)
        aggn = agg * lax.rsqrt(di)[:, None]
        hnet_o[...] = jnp.dot(aggn, wpi[...],
                              preferred_element_type=jnp.float32) + bpi[...]
        agg2 = cp[0] + cp[1]
        out_cf = _ssp(jnp.dot(agg2, wo[...],
                              preferred_element_type=jnp.float32) + bo[...])
        out_sage = (jnp.dot(nf[...], ws[...],
                            preferred_element_type=jnp.float32) + bs[...]
                    + jnp.dot(ng[...], wn[...],
                              preferred_element_type=jnp.float32) + bn[...])
        hnode_o[...] = jnp.maximum(out_cf, out_sage)

    return pl.pallas_call(
        body,
        grid=(G,),
        in_specs=[
            pl.BlockSpec((NC, BR, D), lambda i: (0, i, 0)),
            pl.BlockSpec((NC, BR, D), lambda i: (0, i, 0)),
            pl.BlockSpec((BR, D), lambda i: (i, 0)),
            pl.BlockSpec((BR, D), lambda i: (i, 0)),
            pl.BlockSpec((NC, BR, 1), lambda i: (0, i, 0)),
            pl.BlockSpec((D, D), lambda i: (0, 0)),
            pl.BlockSpec((D,), lambda i: (0,)),
            pl.BlockSpec((D, D), lambda i: (0, 0)),
            pl.BlockSpec((D,), lambda i: (0,)),
            pl.BlockSpec((D, D), lambda i: (0, 0)),
            pl.BlockSpec((D,), lambda i: (0,)),
            pl.BlockSpec((D, D), lambda i: (0, 0)),
            pl.BlockSpec((D,), lambda i: (0,)),
        ],
        out_specs=[
            pl.BlockSpec((BR, D), lambda i: (i, 0)),
            pl.BlockSpec((BR, D), lambda i: (i, 0)),
        ],
        out_shape=[
            jax.ShapeDtypeStruct((N, D), jnp.float32),
            jax.ShapeDtypeStruct((N, D), jnp.float32),
        ],
    )(pins_parts, cf_parts, neigh, node_feat, deg_in_parts,
      W_pins, b_pins, W_o, b_o, W_self, b_self, W_neigh, b_neigh)


def kernel(node_feat, net_feat, pin_feat, edge_feat,
           pins_edge_index, pinned_edge_index, near_edge_index,
           W_gw, b_gw, W_pins, b_pins, W_e1, b_e1, W_e2, b_e2,
           W_n, b_n, W_o, b_o, W_pool, b_pool, W_self, b_self,
           W_neigh, b_neigh):
    src1, dst1 = pins_edge_index[0], pins_edge_index[1]
    src2, dst2 = pinned_edge_index[0], pinned_edge_index[1]
    src3, dst3 = near_edge_index[0], near_edge_index[1]

    he, ew2 = _tc_edge_dense(pin_feat, edge_feat, W_e1, b_e1, W_e2, b_e2,
                             W_gw, b_gw)
    ew = jnp.reshape(ew2, (E,))
    hv, feat_src = _tc_node_dense(net_feat, node_feat, W_n, b_n,
                                  W_pool, b_pool)

    degs = _sc_degrees(src1, dst1)
    feat = _tc_scale(node_feat, degs[:, 0][..., None])

    pins_parts = _sc_segsum(src1, dst1, feat)
    cf_parts = _sc_segsum(src2, dst2, hv, wtab=he)
    neigh = jnp.reshape(_sc_segmax(src3, dst3, ew, feat_src), (N, D))

    h_node, h_net = _tc_final(pins_parts, cf_parts, neigh, node_feat,
                              degs[:, 1][..., None], W_pins, b_pins, W_o, b_o,
                              W_self, b_self, W_neigh, b_neigh)
    return h_node, h_net
